# Initial kernel scaffold; baseline (speedup 1.0000x reference)
import jax
import jax.numpy as jnp
from jax.experimental import pallas as pl


def kernel(x, edge_index, batch_size, W_gat, a_src, a_dst, b_gat, W_gcn, b_gcn, ln_gamma, ln_beta, W_fc2, b_fc2):
    raise NotImplementedError("write your pallas kernel here")


# speedup vs baseline: 28.4250x; 28.4250x over previous
import functools

import jax
import jax.numpy as jnp
from jax import lax
from jax.experimental import pallas as pl
from jax.experimental.pallas import tpu as pltpu
from jax.experimental.pallas import tpu_sc as plsc

N = 10000
E = 320000
D = 128
DP = 144
DC = 16
NC = 2
NS = 16
NT = NC * NS
EPT = E // NT
CK = 80
NCH = EPT // CK
GC = 25
NB = 10
NPB = N // NB
NSTR_OFF = 624
NSTR_SZ = 640


def _tc1_body(x_ref, wg_ref, av_ref, aw_ref, xl_ref, s_ref, d_ref):
    xl = jnp.dot(x_ref[...], wg_ref[...], preferred_element_type=jnp.float32)
    xl_ref[...] = xl
    s_ref[...] = jnp.dot(xl, av_ref[...], preferred_element_type=jnp.float32)
    d_ref[...] = jnp.dot(xl, aw_ref[...], preferred_element_type=jnp.float32)


def _tc1(x, wg, av, aw):
    return pl.pallas_call(
        _tc1_body,
        grid=(NB,),
        in_specs=[
            pl.BlockSpec((NPB, D), lambda b: (b, 0)),
            pl.BlockSpec((D, D), lambda b: (0, 0)),
            pl.BlockSpec((D, 1), lambda b: (0, 0)),
            pl.BlockSpec((D, 1), lambda b: (0, 0)),
        ],
        out_specs=[
            pl.BlockSpec((NPB, D), lambda b: (b, 0)),
            pl.BlockSpec((NPB, 1), lambda b: (b, 0)),
            pl.BlockSpec((NPB, 1), lambda b: (b, 0)),
        ],
        out_shape=[
            jax.ShapeDtypeStruct((N, D), jnp.float32),
            jax.ShapeDtypeStruct((N, 1), jnp.float32),
            jax.ShapeDtypeStruct((N, 1), jnp.float32),
        ],
    )(x, wg, av, aw)


def _sc1_body(xlp_hbm, src_hbm, dst_hbm, adst_hbm,
              acc_out, ee_out,
              acc_sh, adst_v, src_g, dst_g, ee_g, gbuf):
    cid = lax.axis_index("c")
    sid = lax.axis_index("s")
    wid = sid * NC + cid
    r0 = sid * NSTR_OFF
    iota16 = lax.iota(jnp.int32, 16)

    def zrow(r, rc):
        for j in range(DP // 16):
            gbuf[r, pl.ds(j * 16, 16)] = jnp.zeros((16,), jnp.float32)
        return rc

    lax.fori_loop(0, CK, zrow, 0)
    for p in range(NSTR_SZ // CK):
        pltpu.sync_copy(gbuf, acc_sh.at[pl.ds(r0 + p * CK, CK)])
    pltpu.sync_copy(adst_hbm, adst_v)
    plsc.subcore_barrier()

    def group(gg, gcarry):
        pltpu.sync_copy(src_hbm.at[wid, pl.ds(gg * GC, GC)], src_g)
        pltpu.sync_copy(dst_hbm.at[wid, pl.ds(gg * GC, GC)], dst_g)

        def chunk(cc, carry):
            pltpu.sync_copy(xlp_hbm.at[src_g.at[cc]], gbuf)
            for g in range(CK // 16):
                ridx = g * 16 + iota16
                av = plsc.load_gather(gbuf, [ridx, jnp.full((16,), D + 1, jnp.int32)])
                di = dst_g[cc, pl.ds(g * 16, 16)]
                bv = plsc.load_gather(adst_v, [di])
                sm = av + bv
                e = jnp.maximum(sm, 0.0) + 0.2 * jnp.minimum(sm, 0.0)
                ee = jnp.exp(e)
                ee_g[cc, pl.ds(g * 16, 16)] = ee
                for l in range(16):
                    sv = jnp.full((16,), ee[l], jnp.float32)
                    for j in range(DP // 16):
                        gbuf[g * 16 + l, pl.ds(j * 16, 16)] = (
                            gbuf[g * 16 + l, pl.ds(j * 16, 16)] * sv)
            pltpu.sync_copy(gbuf, acc_sh.at[dst_g.at[cc]], add=True)
            return carry

        lax.fori_loop(0, GC, chunk, 0)
        pltpu.sync_copy(ee_g, ee_out.at[wid, pl.ds(gg * GC, GC)])
        return gcarry

    lax.fori_loop(0, NCH // GC, group, 0)
    plsc.subcore_barrier()
    for p in range(NSTR_SZ // CK):
        pltpu.sync_copy(acc_sh.at[pl.ds(r0 + p * CK, CK)], gbuf)
        pltpu.sync_copy(gbuf, acc_out.at[cid, pl.ds(r0 + p * CK, CK)])


_sc1 = functools.partial(
    pl.kernel,
    out_type=(
        jax.ShapeDtypeStruct((NC, N, DP), jnp.float32),
        jax.ShapeDtypeStruct((NT, NCH, CK), jnp.float32),
    ),
    mesh=plsc.VectorSubcoreMesh(core_axis_name="c", subcore_axis_name="s"),
    compiler_params=pltpu.CompilerParams(needs_layout_passes=False, use_tc_tiling_on_sc=False),
    scratch_types=[
        pltpu.VMEM_SHARED((N, DP), jnp.float32),
        pltpu.VMEM((N,), jnp.float32),
        pltpu.VMEM((GC, CK), jnp.int32),
        pltpu.VMEM((GC, CK), jnp.int32),
        pltpu.VMEM((GC, CK), jnp.float32),
        pltpu.VMEM((CK, DP), jnp.float32),
    ],
)(_sc1_body)


def _tc2_body(a0_ref, a1_ref, xl_ref, s_ref, d_ref, bg_ref, wg_ref,
              x2p_ref, den_ref, ees_ref, xenc_ref):
    a0 = a0_ref[...]
    a1 = a1_ref[...]
    feat = a0[:, :D] + a1[:, :D]
    dene = a0[:, D:D + 1] + a1[:, D:D + 1]
    s = s_ref[...] + d_ref[...]
    es = jnp.maximum(s, 0.0) + 0.2 * jnp.minimum(s, 0.0)
    ees = jnp.exp(es)
    den = dene + ees
    xl = xl_ref[...]
    gat = (feat + ees * xl) / den + bg_ref[...]
    xenc = jnp.where(gat > 0, gat, jnp.exp(jnp.minimum(gat, 0.0)) - 1.0)
    xenc_ref[...] = xenc
    x2p_ref[...] = jnp.dot(xenc, wg_ref[...], preferred_element_type=jnp.float32)
    den_ref[...] = den
    ees_ref[...] = ees


def _tc2(a0, a1, xl, asrc, adst, bg, wgcnp):
    return pl.pallas_call(
        _tc2_body,
        grid=(NB,),
        in_specs=[
            pl.BlockSpec((NPB, DP), lambda b: (b, 0)),
            pl.BlockSpec((NPB, DP), lambda b: (b, 0)),
            pl.BlockSpec((NPB, D), lambda b: (b, 0)),
            pl.BlockSpec((NPB, 1), lambda b: (b, 0)),
            pl.BlockSpec((NPB, 1), lambda b: (b, 0)),
            pl.BlockSpec((1, D), lambda b: (0, 0)),
            pl.BlockSpec((D, DC), lambda b: (0, 0)),
        ],
        out_specs=[
            pl.BlockSpec((NPB, DC), lambda b: (b, 0)),
            pl.BlockSpec((NPB, 1), lambda b: (b, 0)),
            pl.BlockSpec((NPB, 1), lambda b: (b, 0)),
            pl.BlockSpec((NPB, D), lambda b: (b, 0)),
        ],
        out_shape=[
            jax.ShapeDtypeStruct((N, DC), jnp.float32),
            jax.ShapeDtypeStruct((N, 1), jnp.float32),
            jax.ShapeDtypeStruct((N, 1), jnp.float32),
            jax.ShapeDtypeStruct((N, D), jnp.float32),
        ],
    )(a0, a1, xl, asrc, adst, bg, wgcnp)


_FCK = 20
_FKB = (N * D // NB) // _FCK


def _fc2_body(x_ref, w_ref, b_ref, out_ref, acc_ref):
    k = pl.program_id(0)

    @pl.when(k == 0)
    def _():
        acc_ref[...] = jnp.zeros((NB, NB), jnp.float32)

    acc_ref[...] += jnp.dot(x_ref[...], w_ref[...],
                            preferred_element_type=jnp.float32)

    @pl.when(k == _FCK - 1)
    def _():
        lg = acc_ref[...] + b_ref[...]
        m = jnp.max(lg, axis=1, keepdims=True)
        p = jnp.exp(lg - m)
        out_ref[...] = p / jnp.sum(p, axis=1, keepdims=True)


def _fc2(xf, wt, b):
    return pl.pallas_call(
        _fc2_body,
        grid=(_FCK,),
        in_specs=[
            pl.BlockSpec((NB, _FKB), lambda k: (0, k)),
            pl.BlockSpec((_FKB, NB), lambda k: (k, 0)),
            pl.BlockSpec((1, NB), lambda k: (0, 0)),
        ],
        out_specs=pl.BlockSpec((NB, NB), lambda k: (0, 0)),
        out_shape=jax.ShapeDtypeStruct((NB, NB), jnp.float32),
        scratch_shapes=[pltpu.VMEM((NB, NB), jnp.float32)],
    )(xf, wt, b)


def _sc2_body(x2p_hbm, src_hbm, dst_hbm, ee_hbm,
              acc_out,
              acc_sh, src_v, dst_v, ee_v, gbuf):
    cid = lax.axis_index("c")
    sid = lax.axis_index("s")
    wid = sid * NC + cid
    r0 = sid * NSTR_OFF

    def zrow(r, rc):
        gbuf[r, pl.ds(0, 16)] = jnp.zeros((16,), jnp.float32)
        return rc

    lax.fori_loop(0, CK, zrow, 0)
    for p in range(NSTR_SZ // CK):
        pltpu.sync_copy(gbuf, acc_sh.at[pl.ds(r0 + p * CK, CK)])
    pltpu.sync_copy(src_hbm.at[wid], src_v)
    pltpu.sync_copy(dst_hbm.at[wid], dst_v)
    pltpu.sync_copy(ee_hbm.at[wid], ee_v)
    plsc.subcore_barrier()

    def chunk(c, carry):
        pltpu.sync_copy(x2p_hbm.at[src_v.at[c]], gbuf)

        def rowgrp(g, rc):
            ev = ee_v[c, pl.ds(g * 16, 16)]
            for l in range(16):
                sv = jnp.full((16,), ev[l], jnp.float32)
                gbuf[g * 16 + l, pl.ds(0, 16)] = gbuf[g * 16 + l, pl.ds(0, 16)] * sv
            return rc

        lax.fori_loop(0, CK // 16, rowgrp, 0)
        pltpu.sync_copy(gbuf, acc_sh.at[dst_v.at[c]], add=True)
        return carry

    lax.fori_loop(0, NCH, chunk, 0)
    plsc.subcore_barrier()
    for p in range(NSTR_SZ // CK):
        pltpu.sync_copy(acc_sh.at[pl.ds(r0 + p * CK, CK)], gbuf)
        pltpu.sync_copy(gbuf, acc_out.at[cid, pl.ds(r0 + p * CK, CK)])


_sc2 = functools.partial(
    pl.kernel,
    out_type=jax.ShapeDtypeStruct((NC, N, DC), jnp.float32),
    mesh=plsc.VectorSubcoreMesh(core_axis_name="c", subcore_axis_name="s"),
    compiler_params=pltpu.CompilerParams(needs_layout_passes=False, use_tc_tiling_on_sc=False),
    scratch_types=[
        pltpu.VMEM_SHARED((N, DC), jnp.float32),
        pltpu.VMEM((NCH, CK), jnp.int32),
        pltpu.VMEM((NCH, CK), jnp.int32),
        pltpu.VMEM((NCH, CK), jnp.float32),
        pltpu.VMEM((CK, DC), jnp.float32),
    ],
)(_sc2_body)


def _tc3_body(g0_ref, g1_ref, x2p_ref, den_ref, ees_ref, bg_ref, gm_ref,
              bt_ref, comm_ref):
    acg = g0_ref[...][:, :8] + g1_ref[...][:, :8]
    x2 = x2p_ref[...][:, :8]
    den = den_ref[...]
    ees = ees_ref[...]
    ges = (acg + ees * x2) / den
    gcn = 0.5 * (ges + x2) + bg_ref[...]
    h = jnp.maximum(gcn, 0.0) + 0.01 * jnp.minimum(gcn, 0.0)
    mu = jnp.mean(h, axis=1, keepdims=True)
    va = jnp.mean((h - mu) ** 2, axis=1, keepdims=True)
    hn = (h - mu) * lax.rsqrt(va + 1e-5) * gm_ref[...] + bt_ref[...]
    m = jnp.max(hn, axis=1, keepdims=True)
    p = jnp.exp(hn - m)
    comm_ref[...] = p / jnp.sum(p, axis=1, keepdims=True)


def _tc3(g0, g1, x2p, den, ees, bg, gm, bt):
    return pl.pallas_call(
        _tc3_body,
        grid=(NB,),
        in_specs=[
            pl.BlockSpec((NPB, DC), lambda b: (b, 0)),
            pl.BlockSpec((NPB, DC), lambda b: (b, 0)),
            pl.BlockSpec((NPB, DC), lambda b: (b, 0)),
            pl.BlockSpec((NPB, 1), lambda b: (b, 0)),
            pl.BlockSpec((NPB, 1), lambda b: (b, 0)),
            pl.BlockSpec((1, 8), lambda b: (0, 0)),
            pl.BlockSpec((1, 8), lambda b: (0, 0)),
            pl.BlockSpec((1, 8), lambda b: (0, 0)),
        ],
        out_specs=pl.BlockSpec((NPB, 8), lambda b: (b, 0)),
        out_shape=jax.ShapeDtypeStruct((N, 8), jnp.float32),
    )(g0, g1, x2p, den, ees, bg, gm, bt)


def kernel(x, edge_index, batch_size, W_gat, a_src, a_dst, b_gat, W_gcn,
           b_gcn, ln_gamma, ln_beta, W_fc2, b_fc2):
    src = edge_index[0].astype(jnp.int32).reshape(NT, NCH, CK)
    dst = edge_index[1].astype(jnp.int32).reshape(NT, NCH, CK)

    x_lin, asrc, adst = _tc1(x, W_gat, a_src.reshape(D, 1), a_dst.reshape(D, 1))

    xlp = jnp.concatenate(
        [x_lin, jnp.ones((N, 1), jnp.float32), asrc,
         jnp.zeros((N, DP - D - 2), jnp.float32)], axis=1)
    acc, ee = _sc1(xlp, src, dst, adst.reshape(N))

    wgcnp = jnp.concatenate([W_gcn, jnp.zeros((D, DC - 8), jnp.float32)], axis=1)
    x2p, den, ees, xenc = _tc2(acc[0], acc[1], x_lin, asrc, adst,
                               b_gat.reshape(1, D), wgcnp)

    x_cls = _fc2(xenc.reshape(NB, NPB * D), W_fc2.T, b_fc2.reshape(1, NB))

    accg = _sc2(x2p, src, dst, ee)

    comm = _tc3(accg[0], accg[1], x2p, den, ees, b_gcn.reshape(1, 8),
                ln_gamma.reshape(1, 8), ln_beta.reshape(1, 8))
    return x_cls, comm.reshape(NB, NPB, 8)


# speedup vs baseline: 39.1013x; 1.3756x over previous
import functools

import jax
import jax.numpy as jnp
from jax import lax
from jax.experimental import pallas as pl
from jax.experimental.pallas import tpu as pltpu
from jax.experimental.pallas import tpu_sc as plsc

N = 10000
E = 320000
D = 128
DP = 144
DC = 16
NC = 2
NS = 16
NT = NC * NS
EPT = E // NT
CK = 80
NCH = EPT // CK
GC = 25
NB = 10
NPB = N // NB
NSTR_OFF = 624
NSTR_SZ = 640


def _tc1_body(x_ref, wg_ref, av_ref, aw_ref, xl_ref, s_ref, d_ref):
    xl = jnp.dot(x_ref[...], wg_ref[...], preferred_element_type=jnp.float32)
    xl_ref[...] = xl
    s_ref[...] = jnp.dot(xl, av_ref[...], preferred_element_type=jnp.float32)
    d_ref[...] = jnp.dot(xl, aw_ref[...], preferred_element_type=jnp.float32)


def _tc1(x, wg, av, aw):
    return pl.pallas_call(
        _tc1_body,
        grid=(NB,),
        in_specs=[
            pl.BlockSpec((NPB, D), lambda b: (b, 0)),
            pl.BlockSpec((D, D), lambda b: (0, 0)),
            pl.BlockSpec((D, 1), lambda b: (0, 0)),
            pl.BlockSpec((D, 1), lambda b: (0, 0)),
        ],
        out_specs=[
            pl.BlockSpec((NPB, D), lambda b: (b, 0)),
            pl.BlockSpec((NPB, 1), lambda b: (b, 0)),
            pl.BlockSpec((NPB, 1), lambda b: (b, 0)),
        ],
        out_shape=[
            jax.ShapeDtypeStruct((N, D), jnp.float32),
            jax.ShapeDtypeStruct((N, 1), jnp.float32),
            jax.ShapeDtypeStruct((N, 1), jnp.float32),
        ],
    )(x, wg, av, aw)


def _sc1_body(xlp_hbm, src_hbm, dst_hbm, adst_hbm,
              acc_out,
              acc_sh, adst_v, src_g, dst_g, gbuf, gsem):
    cid = lax.axis_index("c")
    sid = lax.axis_index("s")
    wid = sid * NC + cid
    r0 = sid * NSTR_OFF
    iota16 = lax.iota(jnp.int32, 16)

    def zrow(r, rc):
        for j in range(DP // 16):
            gbuf[0, r, pl.ds(j * 16, 16)] = jnp.zeros((16,), jnp.float32)
        return rc

    lax.fori_loop(0, CK, zrow, 0)
    for p in range(NSTR_SZ // CK):
        pltpu.sync_copy(gbuf.at[0], acc_sh.at[pl.ds(r0 + p * CK, CK)])
    pltpu.sync_copy(adst_hbm, adst_v)
    plsc.subcore_barrier()

    def gather(cc, b):
        return pltpu.make_async_copy(xlp_hbm.at[src_g.at[cc]], gbuf.at[b],
                                     gsem.at[b])

    def process(cc, b):
        gather(cc, b).wait()
        gv = gbuf.at[b]
        for g in range(CK // 16):
            ridx = g * 16 + iota16
            av = plsc.load_gather(gv, [ridx, jnp.full((16,), D + 1, jnp.int32)])
            di = dst_g[cc, pl.ds(g * 16, 16)]
            bv = plsc.load_gather(adst_v, [di])
            sm = av + bv
            e = jnp.maximum(sm, 0.0) + 0.2 * jnp.minimum(sm, 0.0)
            ee = jnp.exp(e)
            for l in range(16):
                sv = jnp.full((16,), ee[l], jnp.float32)
                for j in range(DP // 16):
                    gbuf[b, g * 16 + l, pl.ds(j * 16, 16)] = (
                        gbuf[b, g * 16 + l, pl.ds(j * 16, 16)] * sv)
        pltpu.sync_copy(gv, acc_sh.at[dst_g.at[cc]], add=True)

    def group(gg, gcarry):
        pltpu.sync_copy(src_hbm.at[wid, pl.ds(gg * GC, GC)], src_g)
        pltpu.sync_copy(dst_hbm.at[wid, pl.ds(gg * GC, GC)], dst_g)
        pltpu.async_copy(xlp_hbm.at[src_g.at[0]], gbuf.at[0], gsem.at[0])
        pltpu.async_copy(xlp_hbm.at[src_g.at[1]], gbuf.at[1], gsem.at[1])

        def chunk(cc, carry):
            b = jnp.bitwise_and(cc, 1)
            process(cc, b)
            nxt = cc + 2

            @pl.when(nxt < GC)
            def _():
                pltpu.async_copy(xlp_hbm.at[src_g.at[nxt]], gbuf.at[b],
                                 gsem.at[b])

            return carry

        lax.fori_loop(0, GC, chunk, 0)
        return gcarry

    lax.fori_loop(0, NCH // GC, group, 0)
    plsc.subcore_barrier()
    for p in range(NSTR_SZ // CK):
        pltpu.sync_copy(acc_sh.at[pl.ds(r0 + p * CK, CK)], gbuf.at[0])
        pltpu.sync_copy(gbuf.at[0], acc_out.at[cid, pl.ds(r0 + p * CK, CK)])


_sc1 = functools.partial(
    pl.kernel,
    out_type=jax.ShapeDtypeStruct((NC, N, DP), jnp.float32),
    mesh=plsc.VectorSubcoreMesh(core_axis_name="c", subcore_axis_name="s"),
    compiler_params=pltpu.CompilerParams(needs_layout_passes=False, use_tc_tiling_on_sc=False),
    scratch_types=[
        pltpu.VMEM_SHARED((N, DP), jnp.float32),
        pltpu.VMEM((N,), jnp.float32),
        pltpu.VMEM((GC, CK), jnp.int32),
        pltpu.VMEM((GC, CK), jnp.int32),
        pltpu.VMEM((2, CK, DP), jnp.float32),
        pltpu.SemaphoreType.DMA((2,)),
    ],
)(_sc1_body)


def _tc2_body(a0_ref, a1_ref, xl_ref, s_ref, d_ref, bg_ref, wg_ref,
              x2p_ref, den_ref, ees_ref, xenc_ref):
    a0 = a0_ref[...]
    a1 = a1_ref[...]
    feat = a0[:, :D] + a1[:, :D]
    dene = a0[:, D:D + 1] + a1[:, D:D + 1]
    s = s_ref[...] + d_ref[...]
    es = jnp.maximum(s, 0.0) + 0.2 * jnp.minimum(s, 0.0)
    ees = jnp.exp(es)
    den = dene + ees
    xl = xl_ref[...]
    gat = (feat + ees * xl) / den + bg_ref[...]
    xenc = jnp.where(gat > 0, gat, jnp.exp(jnp.minimum(gat, 0.0)) - 1.0)
    xenc_ref[...] = xenc
    lane = lax.broadcasted_iota(jnp.int32, (NPB, DC), 1)
    asrc_l8 = jnp.where(lane == 8, s_ref[...], 0.0)
    x2p_ref[...] = (jnp.dot(xenc, wg_ref[...], preferred_element_type=jnp.float32)
                    + asrc_l8)
    den_ref[...] = den
    ees_ref[...] = ees


def _tc2(a0, a1, xl, asrc, adst, bg, wgcnp):
    return pl.pallas_call(
        _tc2_body,
        grid=(NB,),
        in_specs=[
            pl.BlockSpec((NPB, DP), lambda b: (b, 0)),
            pl.BlockSpec((NPB, DP), lambda b: (b, 0)),
            pl.BlockSpec((NPB, D), lambda b: (b, 0)),
            pl.BlockSpec((NPB, 1), lambda b: (b, 0)),
            pl.BlockSpec((NPB, 1), lambda b: (b, 0)),
            pl.BlockSpec((1, D), lambda b: (0, 0)),
            pl.BlockSpec((D, DC), lambda b: (0, 0)),
        ],
        out_specs=[
            pl.BlockSpec((NPB, DC), lambda b: (b, 0)),
            pl.BlockSpec((NPB, 1), lambda b: (b, 0)),
            pl.BlockSpec((NPB, 1), lambda b: (b, 0)),
            pl.BlockSpec((NPB, D), lambda b: (b, 0)),
        ],
        out_shape=[
            jax.ShapeDtypeStruct((N, DC), jnp.float32),
            jax.ShapeDtypeStruct((N, 1), jnp.float32),
            jax.ShapeDtypeStruct((N, 1), jnp.float32),
            jax.ShapeDtypeStruct((N, D), jnp.float32),
        ],
    )(a0, a1, xl, asrc, adst, bg, wgcnp)


_FCK = 20
_FKB = (N * D // NB) // _FCK


def _fc2_body(x_ref, w_ref, b_ref, out_ref, acc_ref):
    k = pl.program_id(0)

    @pl.when(k == 0)
    def _():
        acc_ref[...] = jnp.zeros((NB, NB), jnp.float32)

    acc_ref[...] += jnp.dot(x_ref[...], w_ref[...],
                            preferred_element_type=jnp.float32)

    @pl.when(k == _FCK - 1)
    def _():
        lg = acc_ref[...] + b_ref[...]
        m = jnp.max(lg, axis=1, keepdims=True)
        p = jnp.exp(lg - m)
        out_ref[...] = p / jnp.sum(p, axis=1, keepdims=True)


def _fc2(xf, wt, b):
    return pl.pallas_call(
        _fc2_body,
        grid=(_FCK,),
        in_specs=[
            pl.BlockSpec((NB, _FKB), lambda k: (0, k)),
            pl.BlockSpec((_FKB, NB), lambda k: (k, 0)),
            pl.BlockSpec((1, NB), lambda k: (0, 0)),
        ],
        out_specs=pl.BlockSpec((NB, NB), lambda k: (0, 0)),
        out_shape=jax.ShapeDtypeStruct((NB, NB), jnp.float32),
        scratch_shapes=[pltpu.VMEM((NB, NB), jnp.float32)],
    )(xf, wt, b)


def _sc2_body(x2p_hbm, src_hbm, dst_hbm, adst_hbm,
              acc_out,
              acc_sh, adst_v, src_g, dst_g, gbuf, gsem):
    cid = lax.axis_index("c")
    sid = lax.axis_index("s")
    wid = sid * NC + cid
    r0 = sid * NSTR_OFF
    iota16 = lax.iota(jnp.int32, 16)

    def zrow(r, rc):
        gbuf[0, r, pl.ds(0, 16)] = jnp.zeros((16,), jnp.float32)
        return rc

    lax.fori_loop(0, CK, zrow, 0)
    for p in range(NSTR_SZ // CK):
        pltpu.sync_copy(gbuf.at[0], acc_sh.at[pl.ds(r0 + p * CK, CK)])
    pltpu.sync_copy(adst_hbm, adst_v)
    plsc.subcore_barrier()

    def gather(cc, b):
        return pltpu.make_async_copy(x2p_hbm.at[src_g.at[cc]], gbuf.at[b],
                                     gsem.at[b])

    def process(cc, b):
        gather(cc, b).wait()
        gv = gbuf.at[b]
        for g in range(CK // 16):
            ridx = g * 16 + iota16
            av = plsc.load_gather(gv, [ridx, jnp.full((16,), 8, jnp.int32)])
            di = dst_g[cc, pl.ds(g * 16, 16)]
            bv = plsc.load_gather(adst_v, [di])
            sm = av + bv
            e = jnp.maximum(sm, 0.0) + 0.2 * jnp.minimum(sm, 0.0)
            ee = jnp.exp(e)
            for l in range(16):
                sv = jnp.full((16,), ee[l], jnp.float32)
                gbuf[b, g * 16 + l, pl.ds(0, 16)] = (
                    gbuf[b, g * 16 + l, pl.ds(0, 16)] * sv)
        pltpu.sync_copy(gv, acc_sh.at[dst_g.at[cc]], add=True)

    def group(gg, gcarry):
        pltpu.sync_copy(src_hbm.at[wid, pl.ds(gg * GC, GC)], src_g)
        pltpu.sync_copy(dst_hbm.at[wid, pl.ds(gg * GC, GC)], dst_g)
        pltpu.async_copy(x2p_hbm.at[src_g.at[0]], gbuf.at[0], gsem.at[0])
        pltpu.async_copy(x2p_hbm.at[src_g.at[1]], gbuf.at[1], gsem.at[1])

        def chunk(cc, carry):
            b = jnp.bitwise_and(cc, 1)
            process(cc, b)
            nxt = cc + 2

            @pl.when(nxt < GC)
            def _():
                pltpu.async_copy(x2p_hbm.at[src_g.at[nxt]], gbuf.at[b],
                                 gsem.at[b])

            return carry

        lax.fori_loop(0, GC, chunk, 0)
        return gcarry

    lax.fori_loop(0, NCH // GC, group, 0)
    plsc.subcore_barrier()
    for p in range(NSTR_SZ // CK):
        pltpu.sync_copy(acc_sh.at[pl.ds(r0 + p * CK, CK)], gbuf.at[0])
        pltpu.sync_copy(gbuf.at[0], acc_out.at[cid, pl.ds(r0 + p * CK, CK)])


_sc2 = functools.partial(
    pl.kernel,
    out_type=jax.ShapeDtypeStruct((NC, N, DC), jnp.float32),
    mesh=plsc.VectorSubcoreMesh(core_axis_name="c", subcore_axis_name="s"),
    compiler_params=pltpu.CompilerParams(needs_layout_passes=False, use_tc_tiling_on_sc=False),
    scratch_types=[
        pltpu.VMEM_SHARED((N, DC), jnp.float32),
        pltpu.VMEM((N,), jnp.float32),
        pltpu.VMEM((GC, CK), jnp.int32),
        pltpu.VMEM((GC, CK), jnp.int32),
        pltpu.VMEM((2, CK, DC), jnp.float32),
        pltpu.SemaphoreType.DMA((2,)),
    ],
)(_sc2_body)


def _tc3_body(g0_ref, g1_ref, x2p_ref, den_ref, ees_ref, bg_ref, gm_ref,
              bt_ref, comm_ref):
    acg = g0_ref[...][:, :8] + g1_ref[...][:, :8]
    x2 = x2p_ref[...][:, :8]
    den = den_ref[...]
    ees = ees_ref[...]
    ges = (acg + ees * x2) / den
    gcn = 0.5 * (ges + x2) + bg_ref[...]
    h = jnp.maximum(gcn, 0.0) + 0.01 * jnp.minimum(gcn, 0.0)
    mu = jnp.mean(h, axis=1, keepdims=True)
    va = jnp.mean((h - mu) ** 2, axis=1, keepdims=True)
    hn = (h - mu) * lax.rsqrt(va + 1e-5) * gm_ref[...] + bt_ref[...]
    m = jnp.max(hn, axis=1, keepdims=True)
    p = jnp.exp(hn - m)
    comm_ref[...] = p / jnp.sum(p, axis=1, keepdims=True)


def _tc3(g0, g1, x2p, den, ees, bg, gm, bt):
    return pl.pallas_call(
        _tc3_body,
        grid=(NB,),
        in_specs=[
            pl.BlockSpec((NPB, DC), lambda b: (b, 0)),
            pl.BlockSpec((NPB, DC), lambda b: (b, 0)),
            pl.BlockSpec((NPB, DC), lambda b: (b, 0)),
            pl.BlockSpec((NPB, 1), lambda b: (b, 0)),
            pl.BlockSpec((NPB, 1), lambda b: (b, 0)),
            pl.BlockSpec((1, 8), lambda b: (0, 0)),
            pl.BlockSpec((1, 8), lambda b: (0, 0)),
            pl.BlockSpec((1, 8), lambda b: (0, 0)),
        ],
        out_specs=pl.BlockSpec((NPB, 8), lambda b: (b, 0)),
        out_shape=jax.ShapeDtypeStruct((N, 8), jnp.float32),
    )(g0, g1, x2p, den, ees, bg, gm, bt)


def kernel(x, edge_index, batch_size, W_gat, a_src, a_dst, b_gat, W_gcn,
           b_gcn, ln_gamma, ln_beta, W_fc2, b_fc2):
    src = edge_index[0].astype(jnp.int32).reshape(NT, NCH, CK)
    dst = edge_index[1].astype(jnp.int32).reshape(NT, NCH, CK)

    x_lin, asrc, adst = _tc1(x, W_gat, a_src.reshape(D, 1), a_dst.reshape(D, 1))

    xlp = jnp.concatenate(
        [x_lin, jnp.ones((N, 1), jnp.float32), asrc,
         jnp.zeros((N, DP - D - 2), jnp.float32)], axis=1)
    acc = _sc1(xlp, src, dst, adst.reshape(N))

    wgcnp = jnp.concatenate([W_gcn, jnp.zeros((D, DC - 8), jnp.float32)], axis=1)
    x2p, den, ees, xenc = _tc2(acc[0], acc[1], x_lin, asrc, adst,
                               b_gat.reshape(1, D), wgcnp)

    x_cls = _fc2(xenc.reshape(NB, NPB * D), W_fc2.T, b_fc2.reshape(1, NB))

    accg = _sc2(x2p, src, dst, adst.reshape(N))

    comm = _tc3(accg[0], accg[1], x2p, den, ees, b_gcn.reshape(1, 8),
                ln_gamma.reshape(1, 8), ln_beta.reshape(1, 8))
    return x_cls, comm.reshape(NB, NPB, 8)


# speedup vs baseline: 40.6864x; 1.0405x over previous
import functools

import jax
import jax.numpy as jnp
from jax import lax
from jax.experimental import pallas as pl
from jax.experimental.pallas import tpu as pltpu
from jax.experimental.pallas import tpu_sc as plsc

N = 10000
E = 320000
D = 128
DP = 144
DC = 16
NC = 2
NS = 16
NT = NC * NS
EPT = E // NT
CK = 80
NCH = EPT // CK
GC = 25
NB = 10
NPB = N // NB
NSTR_OFF = 624
NSTR_SZ = 640


def _tc1_body(x_ref, wg_ref, av_ref, aw_ref, xl_ref, s_ref, d_ref):
    xl = jnp.dot(x_ref[...], wg_ref[...], preferred_element_type=jnp.float32)
    xl_ref[...] = xl
    s_ref[...] = jnp.dot(xl, av_ref[...], preferred_element_type=jnp.float32)
    d_ref[...] = jnp.dot(xl, aw_ref[...], preferred_element_type=jnp.float32)


def _tc1(x, wg, av, aw):
    return pl.pallas_call(
        _tc1_body,
        grid=(NB,),
        in_specs=[
            pl.BlockSpec((NPB, D), lambda b: (b, 0)),
            pl.BlockSpec((D, D), lambda b: (0, 0)),
            pl.BlockSpec((D, 1), lambda b: (0, 0)),
            pl.BlockSpec((D, 1), lambda b: (0, 0)),
        ],
        out_specs=[
            pl.BlockSpec((NPB, D), lambda b: (b, 0)),
            pl.BlockSpec((NPB, 1), lambda b: (b, 0)),
            pl.BlockSpec((NPB, 1), lambda b: (b, 0)),
        ],
        out_shape=[
            jax.ShapeDtypeStruct((N, D), jnp.float32),
            jax.ShapeDtypeStruct((N, 1), jnp.float32),
            jax.ShapeDtypeStruct((N, 1), jnp.float32),
        ],
    )(x, wg, av, aw)


def _sc1_body(xlp_hbm, src_hbm, dst_hbm, adst_hbm,
              acc_out,
              acc_sh, adst_v, src_g, dst_g, gbuf, gsem):
    cid = lax.axis_index("c")
    sid = lax.axis_index("s")
    wid = sid * NC + cid
    r0 = sid * NSTR_OFF
    iota16 = lax.iota(jnp.int32, 16)

    def zrow(r, rc):
        for j in range(DP // 16):
            gbuf[0, r, pl.ds(j * 16, 16)] = jnp.zeros((16,), jnp.float32)
        return rc

    lax.fori_loop(0, CK, zrow, 0)
    for p in range(NSTR_SZ // CK):
        pltpu.sync_copy(gbuf.at[0], acc_sh.at[pl.ds(r0 + p * CK, CK)])
    pltpu.sync_copy(adst_hbm, adst_v)
    plsc.subcore_barrier()

    def gather(cc, b):
        return pltpu.make_async_copy(xlp_hbm.at[src_g.at[cc]], gbuf.at[b],
                                     gsem.at[b])

    def process(cc, b):
        gather(cc, b).wait()
        gv = gbuf.at[b]
        for g in range(CK // 16):
            ridx = g * 16 + iota16
            av = plsc.load_gather(gv, [ridx, jnp.full((16,), D + 1, jnp.int32)])
            di = dst_g[cc, pl.ds(g * 16, 16)]
            bv = plsc.load_gather(adst_v, [di])
            sm = av + bv
            e = jnp.maximum(sm, 0.0) + 0.2 * jnp.minimum(sm, 0.0)
            ee = jnp.exp(e)
            for l in range(16):
                sv = jnp.full((16,), ee[l], jnp.float32)
                for j in range(DP // 16):
                    gbuf[b, g * 16 + l, pl.ds(j * 16, 16)] = (
                        gbuf[b, g * 16 + l, pl.ds(j * 16, 16)] * sv)
        pltpu.sync_copy(gv, acc_sh.at[dst_g.at[cc]], add=True)

    def group(gg, gcarry):
        pltpu.sync_copy(src_hbm.at[wid, pl.ds(gg * GC, GC)], src_g)
        pltpu.sync_copy(dst_hbm.at[wid, pl.ds(gg * GC, GC)], dst_g)
        pltpu.async_copy(xlp_hbm.at[src_g.at[0]], gbuf.at[0], gsem.at[0])
        pltpu.async_copy(xlp_hbm.at[src_g.at[1]], gbuf.at[1], gsem.at[1])

        def chunk(cc, carry):
            b = jnp.bitwise_and(cc, 1)
            process(cc, b)
            nxt = cc + 2

            @pl.when(nxt < GC)
            def _():
                pltpu.async_copy(xlp_hbm.at[src_g.at[nxt]], gbuf.at[b],
                                 gsem.at[b])

            return carry

        lax.fori_loop(0, GC, chunk, 0)
        return gcarry

    lax.fori_loop(0, NCH // GC, group, 0)
    plsc.subcore_barrier()
    for p in range(NSTR_SZ // CK):
        pltpu.sync_copy(acc_sh.at[pl.ds(r0 + p * CK, CK)], gbuf.at[0])
        pltpu.sync_copy(gbuf.at[0], acc_out.at[cid, pl.ds(r0 + p * CK, CK)])


_sc1 = functools.partial(
    pl.kernel,
    out_type=jax.ShapeDtypeStruct((NC, N, DP), jnp.float32),
    mesh=plsc.VectorSubcoreMesh(core_axis_name="c", subcore_axis_name="s"),
    compiler_params=pltpu.CompilerParams(needs_layout_passes=False, use_tc_tiling_on_sc=False),
    scratch_types=[
        pltpu.VMEM_SHARED((N, DP), jnp.float32),
        pltpu.VMEM((N,), jnp.float32),
        pltpu.VMEM((GC, CK), jnp.int32),
        pltpu.VMEM((GC, CK), jnp.int32),
        pltpu.VMEM((2, CK, DP), jnp.float32),
        pltpu.SemaphoreType.DMA((2,)),
    ],
)(_sc1_body)


def _tc2_body(a0_ref, a1_ref, xl_ref, s_ref, d_ref, bg_ref, wg_ref,
              x2p_ref, den_ref, ees_ref, xenc_ref):
    a0 = a0_ref[...]
    a1 = a1_ref[...]
    feat = a0[:, :D] + a1[:, :D]
    dene = a0[:, D:D + 1] + a1[:, D:D + 1]
    s = s_ref[...] + d_ref[...]
    es = jnp.maximum(s, 0.0) + 0.2 * jnp.minimum(s, 0.0)
    ees = jnp.exp(es)
    den = dene + ees
    xl = xl_ref[...]
    gat = (feat + ees * xl) / den + bg_ref[...]
    xenc = jnp.where(gat > 0, gat, jnp.exp(jnp.minimum(gat, 0.0)) - 1.0)
    xenc_ref[...] = xenc
    lane = lax.broadcasted_iota(jnp.int32, (NPB, DC), 1)
    asrc_l8 = jnp.where(lane == 8, s_ref[...], 0.0)
    x2p_ref[...] = (jnp.dot(xenc, wg_ref[...], preferred_element_type=jnp.float32)
                    + asrc_l8)
    den_ref[...] = den
    ees_ref[...] = ees


def _tc2(a0, a1, xl, asrc, adst, bg, wgcnp):
    return pl.pallas_call(
        _tc2_body,
        grid=(NB,),
        in_specs=[
            pl.BlockSpec((NPB, DP), lambda b: (b, 0)),
            pl.BlockSpec((NPB, DP), lambda b: (b, 0)),
            pl.BlockSpec((NPB, D), lambda b: (b, 0)),
            pl.BlockSpec((NPB, 1), lambda b: (b, 0)),
            pl.BlockSpec((NPB, 1), lambda b: (b, 0)),
            pl.BlockSpec((1, D), lambda b: (0, 0)),
            pl.BlockSpec((D, DC), lambda b: (0, 0)),
        ],
        out_specs=[
            pl.BlockSpec((NPB, DC), lambda b: (b, 0)),
            pl.BlockSpec((NPB, 1), lambda b: (b, 0)),
            pl.BlockSpec((NPB, 1), lambda b: (b, 0)),
            pl.BlockSpec((NPB, D), lambda b: (b, 0)),
        ],
        out_shape=[
            jax.ShapeDtypeStruct((N, DC), jnp.float32),
            jax.ShapeDtypeStruct((N, 1), jnp.float32),
            jax.ShapeDtypeStruct((N, 1), jnp.float32),
            jax.ShapeDtypeStruct((N, D), jnp.float32),
        ],
    )(a0, a1, xl, asrc, adst, bg, wgcnp)


_FCK = 20
_FKB = (N * D // NB) // _FCK


def _fc2_body(x_ref, w_ref, b_ref, out_ref, acc_ref):
    k = pl.program_id(0)

    @pl.when(k == 0)
    def _():
        acc_ref[...] = jnp.zeros((NB, NB), jnp.float32)

    acc_ref[...] += lax.dot_general(
        x_ref[...], w_ref[...], (((1,), (1,)), ((), ())),
        preferred_element_type=jnp.float32)

    @pl.when(k == _FCK - 1)
    def _():
        lg = acc_ref[...] + b_ref[...]
        m = jnp.max(lg, axis=1, keepdims=True)
        p = jnp.exp(lg - m)
        out_ref[...] = p / jnp.sum(p, axis=1, keepdims=True)


def _fc2(xf, wt, b):
    return pl.pallas_call(
        _fc2_body,
        grid=(_FCK,),
        in_specs=[
            pl.BlockSpec((NB, _FKB), lambda k: (0, k)),
            pl.BlockSpec((NB, _FKB), lambda k: (0, k)),
            pl.BlockSpec((1, NB), lambda k: (0, 0)),
        ],
        out_specs=pl.BlockSpec((NB, NB), lambda k: (0, 0)),
        out_shape=jax.ShapeDtypeStruct((NB, NB), jnp.float32),
        scratch_shapes=[pltpu.VMEM((NB, NB), jnp.float32)],
    )(xf, wt, b)


def _sc2_body(x2p_hbm, src_hbm, dst_hbm, adst_hbm,
              acc_out,
              acc_sh, adst_v, src_g, dst_g, gbuf, gsem):
    cid = lax.axis_index("c")
    sid = lax.axis_index("s")
    wid = sid * NC + cid
    r0 = sid * NSTR_OFF
    iota16 = lax.iota(jnp.int32, 16)

    def zrow(r, rc):
        gbuf[0, r, pl.ds(0, 16)] = jnp.zeros((16,), jnp.float32)
        return rc

    lax.fori_loop(0, CK, zrow, 0)
    for p in range(NSTR_SZ // CK):
        pltpu.sync_copy(gbuf.at[0], acc_sh.at[pl.ds(r0 + p * CK, CK)])
    pltpu.sync_copy(adst_hbm, adst_v)
    plsc.subcore_barrier()

    def gather(cc, b):
        return pltpu.make_async_copy(x2p_hbm.at[src_g.at[cc]], gbuf.at[b],
                                     gsem.at[b])

    def process(cc, b):
        gather(cc, b).wait()
        gv = gbuf.at[b]
        for g in range(CK // 16):
            ridx = g * 16 + iota16
            av = plsc.load_gather(gv, [ridx, jnp.full((16,), 8, jnp.int32)])
            di = dst_g[cc, pl.ds(g * 16, 16)]
            bv = plsc.load_gather(adst_v, [di])
            sm = av + bv
            e = jnp.maximum(sm, 0.0) + 0.2 * jnp.minimum(sm, 0.0)
            ee = jnp.exp(e)
            for l in range(16):
                sv = jnp.full((16,), ee[l], jnp.float32)
                gbuf[b, g * 16 + l, pl.ds(0, 16)] = (
                    gbuf[b, g * 16 + l, pl.ds(0, 16)] * sv)
        pltpu.sync_copy(gv, acc_sh.at[dst_g.at[cc]], add=True)

    def group(gg, gcarry):
        pltpu.sync_copy(src_hbm.at[wid, pl.ds(gg * GC, GC)], src_g)
        pltpu.sync_copy(dst_hbm.at[wid, pl.ds(gg * GC, GC)], dst_g)
        pltpu.async_copy(x2p_hbm.at[src_g.at[0]], gbuf.at[0], gsem.at[0])
        pltpu.async_copy(x2p_hbm.at[src_g.at[1]], gbuf.at[1], gsem.at[1])

        def chunk(cc, carry):
            b = jnp.bitwise_and(cc, 1)
            process(cc, b)
            nxt = cc + 2

            @pl.when(nxt < GC)
            def _():
                pltpu.async_copy(x2p_hbm.at[src_g.at[nxt]], gbuf.at[b],
                                 gsem.at[b])

            return carry

        lax.fori_loop(0, GC, chunk, 0)
        return gcarry

    lax.fori_loop(0, NCH // GC, group, 0)
    plsc.subcore_barrier()
    for p in range(NSTR_SZ // CK):
        pltpu.sync_copy(acc_sh.at[pl.ds(r0 + p * CK, CK)], gbuf.at[0])
        pltpu.sync_copy(gbuf.at[0], acc_out.at[cid, pl.ds(r0 + p * CK, CK)])


_sc2 = functools.partial(
    pl.kernel,
    out_type=jax.ShapeDtypeStruct((NC, N, DC), jnp.float32),
    mesh=plsc.VectorSubcoreMesh(core_axis_name="c", subcore_axis_name="s"),
    compiler_params=pltpu.CompilerParams(needs_layout_passes=False, use_tc_tiling_on_sc=False),
    scratch_types=[
        pltpu.VMEM_SHARED((N, DC), jnp.float32),
        pltpu.VMEM((N,), jnp.float32),
        pltpu.VMEM((GC, CK), jnp.int32),
        pltpu.VMEM((GC, CK), jnp.int32),
        pltpu.VMEM((2, CK, DC), jnp.float32),
        pltpu.SemaphoreType.DMA((2,)),
    ],
)(_sc2_body)


def _tc3_body(g0_ref, g1_ref, x2p_ref, den_ref, ees_ref, bg_ref, gm_ref,
              bt_ref, comm_ref):
    acg = g0_ref[...][:, :8] + g1_ref[...][:, :8]
    x2 = x2p_ref[...][:, :8]
    den = den_ref[...]
    ees = ees_ref[...]
    ges = (acg + ees * x2) / den
    gcn = 0.5 * (ges + x2) + bg_ref[...]
    h = jnp.maximum(gcn, 0.0) + 0.01 * jnp.minimum(gcn, 0.0)
    mu = jnp.mean(h, axis=1, keepdims=True)
    va = jnp.mean((h - mu) ** 2, axis=1, keepdims=True)
    hn = (h - mu) * lax.rsqrt(va + 1e-5) * gm_ref[...] + bt_ref[...]
    m = jnp.max(hn, axis=1, keepdims=True)
    p = jnp.exp(hn - m)
    comm_ref[...] = p / jnp.sum(p, axis=1, keepdims=True)


def _tc3(g0, g1, x2p, den, ees, bg, gm, bt):
    return pl.pallas_call(
        _tc3_body,
        grid=(NB,),
        in_specs=[
            pl.BlockSpec((NPB, DC), lambda b: (b, 0)),
            pl.BlockSpec((NPB, DC), lambda b: (b, 0)),
            pl.BlockSpec((NPB, DC), lambda b: (b, 0)),
            pl.BlockSpec((NPB, 1), lambda b: (b, 0)),
            pl.BlockSpec((NPB, 1), lambda b: (b, 0)),
            pl.BlockSpec((1, 8), lambda b: (0, 0)),
            pl.BlockSpec((1, 8), lambda b: (0, 0)),
            pl.BlockSpec((1, 8), lambda b: (0, 0)),
        ],
        out_specs=pl.BlockSpec((NPB, 8), lambda b: (b, 0)),
        out_shape=jax.ShapeDtypeStruct((N, 8), jnp.float32),
    )(g0, g1, x2p, den, ees, bg, gm, bt)


def kernel(x, edge_index, batch_size, W_gat, a_src, a_dst, b_gat, W_gcn,
           b_gcn, ln_gamma, ln_beta, W_fc2, b_fc2):
    src = edge_index[0].astype(jnp.int32).reshape(NT, NCH, CK)
    dst = edge_index[1].astype(jnp.int32).reshape(NT, NCH, CK)

    x_lin, asrc, adst = _tc1(x, W_gat, a_src.reshape(D, 1), a_dst.reshape(D, 1))

    xlp = jnp.concatenate(
        [x_lin, jnp.ones((N, 1), jnp.float32), asrc,
         jnp.zeros((N, DP - D - 2), jnp.float32)], axis=1)
    acc = _sc1(xlp, src, dst, adst.reshape(N))

    wgcnp = jnp.concatenate([W_gcn, jnp.zeros((D, DC - 8), jnp.float32)], axis=1)
    x2p, den, ees, xenc = _tc2(acc[0], acc[1], x_lin, asrc, adst,
                               b_gat.reshape(1, D), wgcnp)

    x_cls = _fc2(xenc.reshape(NB, NPB * D), W_fc2, b_fc2.reshape(1, NB))

    accg = _sc2(x2p, src, dst, adst.reshape(N))

    comm = _tc3(accg[0], accg[1], x2p, den, ees, b_gcn.reshape(1, 8),
                ln_gamma.reshape(1, 8), ln_beta.reshape(1, 8))
    return x_cls, comm.reshape(NB, NPB, 8)


# speedup vs baseline: 42.0851x; 1.0344x over previous
import functools

import jax
import jax.numpy as jnp
from jax import lax
from jax.experimental import pallas as pl
from jax.experimental.pallas import tpu as pltpu
from jax.experimental.pallas import tpu_sc as plsc

N = 10000
E = 320000
D = 128
DP = 144
DC = 16
NC = 2
NS = 16
NT = NC * NS
EPT = E // NT
CK = 80
NCH = EPT // CK
GC = 25
NB = 10
NPB = N // NB
NSTR_OFF = 624
NSTR_SZ = 640


def _tc1_body(x_ref, wg_ref, av_ref, aw_ref, xl_ref, s_ref, d_ref):
    xl = jnp.dot(x_ref[...], wg_ref[...], preferred_element_type=jnp.float32)
    xl_ref[...] = xl
    s_ref[...] = jnp.dot(xl, av_ref[...], preferred_element_type=jnp.float32)
    d_ref[...] = jnp.dot(xl, aw_ref[...], preferred_element_type=jnp.float32)


def _tc1(x, wg, av, aw):
    return pl.pallas_call(
        _tc1_body,
        grid=(NB,),
        in_specs=[
            pl.BlockSpec((NPB, D), lambda b: (b, 0)),
            pl.BlockSpec((D, D), lambda b: (0, 0)),
            pl.BlockSpec((D, 1), lambda b: (0, 0)),
            pl.BlockSpec((D, 1), lambda b: (0, 0)),
        ],
        out_specs=[
            pl.BlockSpec((NPB, D), lambda b: (b, 0)),
            pl.BlockSpec((NPB, 1), lambda b: (b, 0)),
            pl.BlockSpec((NPB, 1), lambda b: (b, 0)),
        ],
        out_shape=[
            jax.ShapeDtypeStruct((N, D), jnp.float32),
            jax.ShapeDtypeStruct((N, 1), jnp.float32),
            jax.ShapeDtypeStruct((N, 1), jnp.float32),
        ],
    )(x, wg, av, aw)


def _sc1_body(xlp_hbm, src_hbm, dst_hbm, adst_hbm,
              acc_out,
              acc_sh, adst_v, src_g, dst_g, gbuf, gsem):
    cid = lax.axis_index("c")
    sid = lax.axis_index("s")
    wid = sid * NC + cid
    r0 = sid * NSTR_OFF
    iota16 = lax.iota(jnp.int32, 16)

    def zrow(r, rc):
        for j in range(DP // 16):
            gbuf[0, r, pl.ds(j * 16, 16)] = jnp.zeros((16,), jnp.float32)
        return rc

    lax.fori_loop(0, CK, zrow, 0)
    for p in range(NSTR_SZ // CK):
        pltpu.sync_copy(gbuf.at[0], acc_sh.at[pl.ds(r0 + p * CK, CK)])
    pltpu.sync_copy(adst_hbm, adst_v)
    plsc.subcore_barrier()

    def gather(cc, b):
        return pltpu.make_async_copy(xlp_hbm.at[src_g.at[cc]], gbuf.at[b],
                                     gsem.at[b])

    def process(cc, b):
        gather(cc, b).wait()
        gv = gbuf.at[b]
        for g in range(CK // 16):
            ridx = g * 16 + iota16
            av = plsc.load_gather(gv, [ridx, jnp.full((16,), D + 1, jnp.int32)])
            di = dst_g[cc, pl.ds(g * 16, 16)]
            bv = plsc.load_gather(adst_v, [di])
            sm = av + bv
            e = jnp.maximum(sm, 0.0) + 0.2 * jnp.minimum(sm, 0.0)
            ee = jnp.exp(e)
            for l in range(16):
                sv = jnp.full((16,), ee[l], jnp.float32)
                for j in range(DP // 16):
                    gbuf[b, g * 16 + l, pl.ds(j * 16, 16)] = (
                        gbuf[b, g * 16 + l, pl.ds(j * 16, 16)] * sv)
        pltpu.sync_copy(gv, acc_sh.at[dst_g.at[cc]], add=True)

    def group(gg, gcarry):
        pltpu.sync_copy(src_hbm.at[wid, pl.ds(gg * GC, GC)], src_g)
        pltpu.sync_copy(dst_hbm.at[wid, pl.ds(gg * GC, GC)], dst_g)
        pltpu.async_copy(xlp_hbm.at[src_g.at[0]], gbuf.at[0], gsem.at[0])
        pltpu.async_copy(xlp_hbm.at[src_g.at[1]], gbuf.at[1], gsem.at[1])

        def chunk(cc, carry):
            b = jnp.bitwise_and(cc, 1)
            process(cc, b)
            nxt = cc + 2

            @pl.when(nxt < GC)
            def _():
                pltpu.async_copy(xlp_hbm.at[src_g.at[nxt]], gbuf.at[b],
                                 gsem.at[b])

            return carry

        lax.fori_loop(0, GC, chunk, 0)
        return gcarry

    lax.fori_loop(0, NCH // GC, group, 0)
    plsc.subcore_barrier()
    for p in range(NSTR_SZ // CK):
        pltpu.sync_copy(acc_sh.at[pl.ds(r0 + p * CK, CK)], gbuf.at[0])
        pltpu.sync_copy(gbuf.at[0], acc_out.at[cid, pl.ds(r0 + p * CK, CK)])


_sc1 = functools.partial(
    pl.kernel,
    out_type=jax.ShapeDtypeStruct((NC, N, DP), jnp.float32),
    mesh=plsc.VectorSubcoreMesh(core_axis_name="c", subcore_axis_name="s"),
    compiler_params=pltpu.CompilerParams(needs_layout_passes=False, use_tc_tiling_on_sc=False),
    scratch_types=[
        pltpu.VMEM_SHARED((N, DP), jnp.float32),
        pltpu.VMEM((N,), jnp.float32),
        pltpu.VMEM((GC, CK), jnp.int32),
        pltpu.VMEM((GC, CK), jnp.int32),
        pltpu.VMEM((2, CK, DP), jnp.float32),
        pltpu.SemaphoreType.DMA((2,)),
    ],
)(_sc1_body)


def _tc2_body(a0_ref, a1_ref, xl_ref, s_ref, d_ref, bg_ref, wg_ref,
              x2p_ref, den_ref, ees_ref, xenc_ref):
    a0 = a0_ref[...]
    a1 = a1_ref[...]
    feat = a0[:, :D] + a1[:, :D]
    dene = a0[:, D:D + 1] + a1[:, D:D + 1]
    s = s_ref[...] + d_ref[...]
    es = jnp.maximum(s, 0.0) + 0.2 * jnp.minimum(s, 0.0)
    ees = jnp.exp(es)
    den = dene + ees
    xl = xl_ref[...]
    gat = (feat + ees * xl) / den + bg_ref[...]
    xenc = jnp.where(gat > 0, gat, jnp.exp(jnp.minimum(gat, 0.0)) - 1.0)
    xenc_ref[...] = xenc
    lane = lax.broadcasted_iota(jnp.int32, (NPB, DC), 1)
    asrc_l8 = jnp.where(lane == 8, s_ref[...], 0.0)
    x2p_ref[...] = (jnp.dot(xenc, wg_ref[...], preferred_element_type=jnp.float32)
                    + asrc_l8)
    den_ref[...] = den
    ees_ref[...] = ees


def _tc2(a0, a1, xl, asrc, adst, bg, wgcnp):
    return pl.pallas_call(
        _tc2_body,
        grid=(NB,),
        in_specs=[
            pl.BlockSpec((NPB, DP), lambda b: (b, 0)),
            pl.BlockSpec((NPB, DP), lambda b: (b, 0)),
            pl.BlockSpec((NPB, D), lambda b: (b, 0)),
            pl.BlockSpec((NPB, 1), lambda b: (b, 0)),
            pl.BlockSpec((NPB, 1), lambda b: (b, 0)),
            pl.BlockSpec((1, D), lambda b: (0, 0)),
            pl.BlockSpec((D, DC), lambda b: (0, 0)),
        ],
        out_specs=[
            pl.BlockSpec((NPB, DC), lambda b: (b, 0)),
            pl.BlockSpec((NPB, 1), lambda b: (b, 0)),
            pl.BlockSpec((NPB, 1), lambda b: (b, 0)),
            pl.BlockSpec((NPB, D), lambda b: (b, 0)),
        ],
        out_shape=[
            jax.ShapeDtypeStruct((N, DC), jnp.float32),
            jax.ShapeDtypeStruct((N, 1), jnp.float32),
            jax.ShapeDtypeStruct((N, 1), jnp.float32),
            jax.ShapeDtypeStruct((N, D), jnp.float32),
        ],
    )(a0, a1, xl, asrc, adst, bg, wgcnp)


_FCK = 20
_FKB = (N * D // NB) // _FCK


def _fc2_body(x_ref, w_ref, b_ref, out_ref, acc_ref):
    k = pl.program_id(0)

    @pl.when(k == 0)
    def _():
        acc_ref[...] = jnp.zeros((NB, NB), jnp.float32)

    acc_ref[...] += lax.dot_general(
        x_ref[...], w_ref[...], (((1,), (1,)), ((), ())),
        preferred_element_type=jnp.float32)

    @pl.when(k == _FCK - 1)
    def _():
        lg = acc_ref[...] + b_ref[...]
        m = jnp.max(lg, axis=1, keepdims=True)
        p = jnp.exp(lg - m)
        out_ref[...] = p / jnp.sum(p, axis=1, keepdims=True)


def _fc2(xf, wt, b):
    return pl.pallas_call(
        _fc2_body,
        grid=(_FCK,),
        in_specs=[
            pl.BlockSpec((NB, _FKB), lambda k: (0, k)),
            pl.BlockSpec((NB, _FKB), lambda k: (0, k)),
            pl.BlockSpec((1, NB), lambda k: (0, 0)),
        ],
        out_specs=pl.BlockSpec((NB, NB), lambda k: (0, 0)),
        out_shape=jax.ShapeDtypeStruct((NB, NB), jnp.float32),
        scratch_shapes=[pltpu.VMEM((NB, NB), jnp.float32)],
    )(xf, wt, b)


def _sc2_body(x2p_hbm, src_hbm, dst_hbm, adst_hbm,
              acc_out,
              acc_sh, adst_v, src_g, dst_g, gbuf, gsem):
    cid = lax.axis_index("c")
    sid = lax.axis_index("s")
    wid = sid * NC + cid
    r0 = sid * NSTR_OFF
    iota16 = lax.iota(jnp.int32, 16)

    def zrow(r, rc):
        gbuf[0, r, pl.ds(0, 16)] = jnp.zeros((16,), jnp.float32)
        return rc

    lax.fori_loop(0, CK, zrow, 0)
    for p in range(NSTR_SZ // CK):
        pltpu.sync_copy(gbuf.at[0], acc_sh.at[pl.ds(r0 + p * CK, CK)])
    pltpu.sync_copy(adst_hbm, adst_v)
    plsc.subcore_barrier()

    def gather(cc, b):
        return pltpu.make_async_copy(x2p_hbm.at[src_g.at[cc]], gbuf.at[b],
                                     gsem.at[b])

    def process(cc, b):
        gather(cc, b).wait()
        gv = gbuf.at[b]
        for g in range(CK // 16):
            ridx = g * 16 + iota16
            av = plsc.load_gather(gv, [ridx, jnp.full((16,), 8, jnp.int32)])
            di = dst_g[cc, pl.ds(g * 16, 16)]
            bv = plsc.load_gather(adst_v, [di])
            sm = av + bv
            e = jnp.maximum(sm, 0.0) + 0.2 * jnp.minimum(sm, 0.0)
            ee = jnp.exp(e)
            for l in range(16):
                sv = jnp.full((16,), ee[l], jnp.float32)
                gbuf[b, g * 16 + l, pl.ds(0, 16)] = (
                    gbuf[b, g * 16 + l, pl.ds(0, 16)] * sv)
        pltpu.sync_copy(gv, acc_sh.at[dst_g.at[cc]], add=True)

    def group(gg, gcarry):
        pltpu.sync_copy(src_hbm.at[wid, pl.ds(gg * GC, GC)], src_g)
        pltpu.sync_copy(dst_hbm.at[wid, pl.ds(gg * GC, GC)], dst_g)
        pltpu.async_copy(x2p_hbm.at[src_g.at[0]], gbuf.at[0], gsem.at[0])
        pltpu.async_copy(x2p_hbm.at[src_g.at[1]], gbuf.at[1], gsem.at[1])
        pltpu.async_copy(x2p_hbm.at[src_g.at[2]], gbuf.at[2], gsem.at[2])

        def chunk(cc, carry):
            b = lax.rem(cc, 3)
            process(cc, b)
            nxt = cc + 3

            @pl.when(nxt < GC)
            def _():
                pltpu.async_copy(x2p_hbm.at[src_g.at[nxt]], gbuf.at[b],
                                 gsem.at[b])

            return carry

        lax.fori_loop(0, GC, chunk, 0)
        return gcarry

    lax.fori_loop(0, NCH // GC, group, 0)
    plsc.subcore_barrier()
    for p in range(NSTR_SZ // CK):
        pltpu.sync_copy(acc_sh.at[pl.ds(r0 + p * CK, CK)], gbuf.at[0])
        pltpu.sync_copy(gbuf.at[0], acc_out.at[cid, pl.ds(r0 + p * CK, CK)])


_sc2 = functools.partial(
    pl.kernel,
    out_type=jax.ShapeDtypeStruct((NC, N, DC), jnp.float32),
    mesh=plsc.VectorSubcoreMesh(core_axis_name="c", subcore_axis_name="s"),
    compiler_params=pltpu.CompilerParams(needs_layout_passes=False, use_tc_tiling_on_sc=False),
    scratch_types=[
        pltpu.VMEM_SHARED((N, DC), jnp.float32),
        pltpu.VMEM((N,), jnp.float32),
        pltpu.VMEM((GC, CK), jnp.int32),
        pltpu.VMEM((GC, CK), jnp.int32),
        pltpu.VMEM((3, CK, DC), jnp.float32),
        pltpu.SemaphoreType.DMA((3,)),
    ],
)(_sc2_body)


def _tc3_body(g0_ref, g1_ref, x2p_ref, den_ref, ees_ref, bg_ref, gm_ref,
              bt_ref, comm_ref):
    acg = g0_ref[...][:, :8] + g1_ref[...][:, :8]
    x2 = x2p_ref[...][:, :8]
    den = den_ref[...]
    ees = ees_ref[...]
    ges = (acg + ees * x2) / den
    gcn = 0.5 * (ges + x2) + bg_ref[...]
    h = jnp.maximum(gcn, 0.0) + 0.01 * jnp.minimum(gcn, 0.0)
    mu = jnp.mean(h, axis=1, keepdims=True)
    va = jnp.mean((h - mu) ** 2, axis=1, keepdims=True)
    hn = (h - mu) * lax.rsqrt(va + 1e-5) * gm_ref[...] + bt_ref[...]
    m = jnp.max(hn, axis=1, keepdims=True)
    p = jnp.exp(hn - m)
    comm_ref[...] = p / jnp.sum(p, axis=1, keepdims=True)


def _tc3(g0, g1, x2p, den, ees, bg, gm, bt):
    return pl.pallas_call(
        _tc3_body,
        grid=(NB,),
        in_specs=[
            pl.BlockSpec((NPB, DC), lambda b: (b, 0)),
            pl.BlockSpec((NPB, DC), lambda b: (b, 0)),
            pl.BlockSpec((NPB, DC), lambda b: (b, 0)),
            pl.BlockSpec((NPB, 1), lambda b: (b, 0)),
            pl.BlockSpec((NPB, 1), lambda b: (b, 0)),
            pl.BlockSpec((1, 8), lambda b: (0, 0)),
            pl.BlockSpec((1, 8), lambda b: (0, 0)),
            pl.BlockSpec((1, 8), lambda b: (0, 0)),
        ],
        out_specs=pl.BlockSpec((NPB, 8), lambda b: (b, 0)),
        out_shape=jax.ShapeDtypeStruct((N, 8), jnp.float32),
    )(g0, g1, x2p, den, ees, bg, gm, bt)


def kernel(x, edge_index, batch_size, W_gat, a_src, a_dst, b_gat, W_gcn,
           b_gcn, ln_gamma, ln_beta, W_fc2, b_fc2):
    src = edge_index[0].astype(jnp.int32).reshape(NT, NCH, CK)
    dst = edge_index[1].astype(jnp.int32).reshape(NT, NCH, CK)

    x_lin, asrc, adst = _tc1(x, W_gat, a_src.reshape(D, 1), a_dst.reshape(D, 1))

    xlp = jnp.concatenate(
        [x_lin, jnp.ones((N, 1), jnp.float32), asrc,
         jnp.zeros((N, DP - D - 2), jnp.float32)], axis=1)
    acc = _sc1(xlp, src, dst, adst.reshape(N))

    wgcnp = jnp.concatenate([W_gcn, jnp.zeros((D, DC - 8), jnp.float32)], axis=1)
    x2p, den, ees, xenc = _tc2(acc[0], acc[1], x_lin, asrc, adst,
                               b_gat.reshape(1, D), wgcnp)

    x_cls = _fc2(xenc.reshape(NB, NPB * D), W_fc2, b_fc2.reshape(1, NB))

    accg = _sc2(x2p, src, dst, adst.reshape(N))

    comm = _tc3(accg[0], accg[1], x2p, den, ees, b_gcn.reshape(1, 8),
                ln_gamma.reshape(1, 8), ln_beta.reshape(1, 8))
    return x_cls, comm.reshape(NB, NPB, 8)
